# baseline (speedup 1.0000x reference)
import functools

import jax
import jax.numpy as jnp
from jax.experimental import pallas as pl
from jax.experimental.pallas import tpu as pltpu

N = 50000
HID = 32
OUT_CH = 32


def _norm_gelu_body(h_ref, w_ref, b_ref, ms_ref, c_ref, o_ref):
    h = h_ref[...]
    C = c_ref[...]
    mean = jnp.mean(h, axis=0, keepdims=True) @ C
    xc = h - ms_ref[...] * mean
    var = jnp.mean(xc * xc, axis=0, keepdims=True) @ C
    y = w_ref[...] * xc * jax.lax.rsqrt(var + 1e-5) + b_ref[...]
    o_ref[...] = 0.5 * y * (1.0 + jax.lax.erf(y * 0.7071067811865475))


def _group_avg_mat():
    i = jnp.arange(128)
    return jnp.where((i[:, None] % 32) == (i[None, :] % 32), 0.25, 0.0)


def _norm_gelu(h, w, b, ms):
    n = h.shape[0]
    h4 = h.reshape(n // 4, 128)
    tile = lambda v: jnp.tile(v.reshape(1, -1), (1, 4))
    out = pl.pallas_call(
        _norm_gelu_body,
        out_shape=jax.ShapeDtypeStruct(h4.shape, h.dtype),
    )(h4, tile(w), tile(b), tile(ms), _group_avg_mat())
    return out.reshape(n, 32)


def _gatv2(x, src, dst, Wl, Wr, att, bias, heads, ch):
    n = x.shape[0]
    xl = (x @ Wl.T).reshape(n, heads, ch)
    xr = (x @ Wr.T).reshape(n, heads, ch)
    e = jax.nn.leaky_relu(xl[src] + xr[dst], negative_slope=0.2)
    logits = (e * att[None, :, :]).sum(-1)
    m = jax.ops.segment_max(logits, dst, num_segments=n)
    m = jnp.where(jnp.isfinite(m), m, 0.0)
    ex = jnp.exp(logits - m[dst])
    denom = jax.ops.segment_sum(ex, dst, num_segments=n)
    alpha = ex / (denom[dst] + 1e-16)
    out = jax.ops.segment_sum(xl[src] * alpha[..., None], dst, num_segments=n)
    return out.mean(axis=1) + bias


def kernel(x, edge_index, Wl1, Wr1, att1, b1, g1w, g1b, g1m,
           Wl2, Wr2, att2, b2, g2w, g2b, g2m,
           Wl3, Wr3, att3, b3, g3w, g3b, g3m):
    n = x.shape[0]
    loop = jnp.arange(n, dtype=edge_index.dtype)
    src = jnp.concatenate([edge_index[0], loop])
    dst = jnp.concatenate([edge_index[1], loop])

    h = _gatv2(x, src, dst, Wl1, Wr1, att1, b1, 2, HID)
    h = _norm_gelu(h, g1w, g1b, g1m)
    h = _gatv2(h, src, dst, Wl2, Wr2, att2, b2, 2, HID)
    h = _norm_gelu(h, g2w, g2b, g2m)
    h = _gatv2(h, src, dst, Wl3, Wr3, att3, b3, 1, OUT_CH)
    h = _norm_gelu(h, g3w, g3b, g3m)
    return h.mean(axis=0, keepdims=True)


# speedup vs baseline: 4.0795x; 4.0795x over previous
import jax
import jax.numpy as jnp
from jax.experimental import pallas as pl

N = 50000
HID = 32
OUT_CH = 32
EB = 5000


def _norm_gelu_body(h_ref, w_ref, b_ref, ms_ref, c_ref, o_ref):
    h = h_ref[...]
    C = c_ref[...]
    mean = jnp.mean(h, axis=0, keepdims=True) @ C
    xc = h - ms_ref[...] * mean
    var = jnp.mean(xc * xc, axis=0, keepdims=True) @ C
    y = w_ref[...] * xc * jax.lax.rsqrt(var + 1e-5) + b_ref[...]
    o_ref[...] = 0.5 * y * (1.0 + jax.lax.erf(y * 0.7071067811865475))


def _group_avg_mat():
    i = jnp.arange(128)
    return jnp.where((i[:, None] % 32) == (i[None, :] % 32), 0.25, 0.0)


def _norm_gelu(h, w, b, ms):
    n = h.shape[0]
    h4 = h.reshape(n // 4, 128)
    tile = lambda v: jnp.tile(v.reshape(1, -1), (1, 4))
    out = pl.pallas_call(
        _norm_gelu_body,
        out_shape=jax.ShapeDtypeStruct(h4.shape, h.dtype),
    )(h4, tile(w), tile(b), tile(ms), _group_avg_mat())
    return out.reshape(n, 32)


def _logits_body(el_ref, er_ref, att_ref, s_ref, o_ref):
    s = el_ref[...] + er_ref[...]
    f = jnp.where(s >= 0.0, s, 0.2 * s) * att_ref[...]
    o_ref[...] = jnp.dot(f, s_ref[...], preferred_element_type=jnp.float32)


def _edge_logits(el, er, att):
    e, c = el.shape
    heads = att.shape[0]
    seg = jnp.repeat(jnp.eye(heads, dtype=el.dtype), c // heads, axis=0)
    return pl.pallas_call(
        _logits_body,
        grid=(e // EB,),
        in_specs=[
            pl.BlockSpec((EB, c), lambda i: (i, 0)),
            pl.BlockSpec((EB, c), lambda i: (i, 0)),
            pl.BlockSpec((1, c), lambda i: (0, 0)),
            pl.BlockSpec((c, heads), lambda i: (0, 0)),
        ],
        out_specs=pl.BlockSpec((EB, heads), lambda i: (i, 0)),
        out_shape=jax.ShapeDtypeStruct((e, heads), el.dtype),
    )(el, er, att.reshape(1, -1), seg)


def _exp_body(lg_ref, md_ref, o_ref):
    o_ref[...] = jnp.exp(lg_ref[...] - md_ref[...])


def _edge_exp(logits, md):
    e, heads = logits.shape
    return pl.pallas_call(
        _exp_body,
        grid=(e // EB,),
        in_specs=[
            pl.BlockSpec((EB, heads), lambda i: (i, 0)),
            pl.BlockSpec((EB, heads), lambda i: (i, 0)),
        ],
        out_specs=pl.BlockSpec((EB, heads), lambda i: (i, 0)),
        out_shape=jax.ShapeDtypeStruct((e, heads), logits.dtype),
    )(logits, md)


def _msg_body(el_ref, ex_ref, dd_ref, st_ref, o_ref):
    alpha = ex_ref[...] / (dd_ref[...] + 1e-16)
    ab = jnp.dot(alpha, st_ref[...], preferred_element_type=jnp.float32)
    o_ref[...] = el_ref[...] * ab


def _edge_msg(el, ex, dd):
    e, c = el.shape
    heads = ex.shape[1]
    seg_t = jnp.repeat(jnp.eye(heads, dtype=el.dtype), c // heads, axis=1)
    return pl.pallas_call(
        _msg_body,
        grid=(e // EB,),
        in_specs=[
            pl.BlockSpec((EB, c), lambda i: (i, 0)),
            pl.BlockSpec((EB, heads), lambda i: (i, 0)),
            pl.BlockSpec((EB, heads), lambda i: (i, 0)),
            pl.BlockSpec((heads, c), lambda i: (0, 0)),
        ],
        out_specs=pl.BlockSpec((EB, c), lambda i: (i, 0)),
        out_shape=jax.ShapeDtypeStruct((e, c), el.dtype),
    )(el, ex, dd, seg_t)


def _gatv2(x, src, dst, Wl, Wr, att, bias, heads, ch):
    n = x.shape[0]
    xl = x @ Wl.T
    xr = x @ Wr.T
    el = xl[src]
    er = xr[dst]
    logits = _edge_logits(el, er, att)
    m = jax.ops.segment_max(logits, dst, num_segments=n)
    m = jnp.where(jnp.isfinite(m), m, 0.0)
    ex = _edge_exp(logits, m[dst])
    denom = jax.ops.segment_sum(ex, dst, num_segments=n)
    msg = _edge_msg(el, ex, denom[dst])
    out = jax.ops.segment_sum(msg, dst, num_segments=n)
    out = out.reshape(n, heads, ch).mean(axis=1)
    return out + bias


def kernel(x, edge_index, Wl1, Wr1, att1, b1, g1w, g1b, g1m,
           Wl2, Wr2, att2, b2, g2w, g2b, g2m,
           Wl3, Wr3, att3, b3, g3w, g3b, g3m):
    n = x.shape[0]
    loop = jnp.arange(n, dtype=edge_index.dtype)
    src = jnp.concatenate([edge_index[0], loop])
    dst = jnp.concatenate([edge_index[1], loop])

    h = _gatv2(x, src, dst, Wl1, Wr1, att1, b1, 2, HID)
    h = _norm_gelu(h, g1w, g1b, g1m)
    h = _gatv2(h, src, dst, Wl2, Wr2, att2, b2, 2, HID)
    h = _norm_gelu(h, g2w, g2b, g2m)
    h = _gatv2(h, src, dst, Wl3, Wr3, att3, b3, 1, OUT_CH)
    h = _norm_gelu(h, g3w, g3b, g3m)
    return h.mean(axis=0, keepdims=True)


# speedup vs baseline: 4.1399x; 1.0148x over previous
import jax
import jax.numpy as jnp
from jax.experimental import pallas as pl

N = 50000
HID = 32
OUT_CH = 32
EB = 5000


def _norm_gelu_body(h_ref, w_ref, b_ref, ms_ref, c_ref, o_ref):
    h = h_ref[...]
    C = c_ref[...]
    mean = jnp.mean(h, axis=0, keepdims=True) @ C
    xc = h - ms_ref[...] * mean
    var = jnp.mean(xc * xc, axis=0, keepdims=True) @ C
    y = w_ref[...] * xc * jax.lax.rsqrt(var + 1e-5) + b_ref[...]
    o_ref[...] = 0.5 * y * (1.0 + jax.lax.erf(y * 0.7071067811865475))


def _group_avg_mat():
    i = jnp.arange(128)
    return jnp.where((i[:, None] % 32) == (i[None, :] % 32), 0.25, 0.0)


def _norm_gelu(h, w, b, ms):
    n = h.shape[0]
    h4 = h.reshape(n // 4, 128)
    tile = lambda v: jnp.tile(v.reshape(1, -1), (1, 4))
    out = pl.pallas_call(
        _norm_gelu_body,
        out_shape=jax.ShapeDtypeStruct(h4.shape, h.dtype),
    )(h4, tile(w), tile(b), tile(ms), _group_avg_mat())
    return out.reshape(n, 32)


def _logits_body(el_ref, er_ref, att_ref, s_ref, o_ref):
    s = el_ref[...] + er_ref[...]
    f = jnp.where(s >= 0.0, s, 0.2 * s) * att_ref[...]
    o_ref[...] = jnp.dot(f, s_ref[...], preferred_element_type=jnp.float32)


def _edge_logits(el, er, att):
    e, c = el.shape
    heads = att.shape[0]
    seg = jnp.repeat(jnp.eye(heads, dtype=el.dtype), c // heads, axis=0)
    return pl.pallas_call(
        _logits_body,
        grid=(e // EB,),
        in_specs=[
            pl.BlockSpec((EB, c), lambda i: (i, 0)),
            pl.BlockSpec((EB, c), lambda i: (i, 0)),
            pl.BlockSpec((1, c), lambda i: (0, 0)),
            pl.BlockSpec((c, heads), lambda i: (0, 0)),
        ],
        out_specs=pl.BlockSpec((EB, heads), lambda i: (i, 0)),
        out_shape=jax.ShapeDtypeStruct((e, heads), el.dtype),
    )(el, er, att.reshape(1, -1), seg)


def _exp_body(lg_ref, md_ref, o_ref):
    o_ref[...] = jnp.exp(lg_ref[...] - md_ref[...])


def _edge_exp(logits, md):
    e, heads = logits.shape
    return pl.pallas_call(
        _exp_body,
        grid=(e // EB,),
        in_specs=[
            pl.BlockSpec((EB, heads), lambda i: (i, 0)),
            pl.BlockSpec((EB, heads), lambda i: (i, 0)),
        ],
        out_specs=pl.BlockSpec((EB, heads), lambda i: (i, 0)),
        out_shape=jax.ShapeDtypeStruct((e, heads), logits.dtype),
    )(logits, md)


def _msg_body(el_ref, ex_ref, dd_ref, st_ref, hm_ref, o_ref):
    alpha = ex_ref[...] / (dd_ref[...] + 1e-16)
    ab = jnp.dot(alpha, st_ref[...], preferred_element_type=jnp.float32)
    o_ref[...] = jnp.dot(el_ref[...] * ab, hm_ref[...],
                         preferred_element_type=jnp.float32)


def _edge_msg(el, ex, dd):
    e, c = el.shape
    heads = ex.shape[1]
    ch = c // heads
    seg_t = jnp.repeat(jnp.eye(heads, dtype=el.dtype), ch, axis=1)
    hmean = jnp.tile(jnp.eye(ch, dtype=el.dtype), (heads, 1)) / heads
    return pl.pallas_call(
        _msg_body,
        grid=(e // EB,),
        in_specs=[
            pl.BlockSpec((EB, c), lambda i: (i, 0)),
            pl.BlockSpec((EB, heads), lambda i: (i, 0)),
            pl.BlockSpec((EB, heads), lambda i: (i, 0)),
            pl.BlockSpec((heads, c), lambda i: (0, 0)),
            pl.BlockSpec((c, ch), lambda i: (0, 0)),
        ],
        out_specs=pl.BlockSpec((EB, ch), lambda i: (i, 0)),
        out_shape=jax.ShapeDtypeStruct((e, ch), el.dtype),
    )(el, ex, dd, seg_t, hmean)


def _gatv2(x, src, dst, Wl, Wr, att, bias, heads, ch):
    n = x.shape[0]
    xl = x @ Wl.T
    xr = x @ Wr.T
    el = xl[src]
    er = xr[dst]
    logits = _edge_logits(el, er, att)
    m = jax.ops.segment_max(logits, dst, num_segments=n)
    m = jnp.where(jnp.isfinite(m), m, 0.0)
    ex = _edge_exp(logits, m[dst])
    denom = jax.ops.segment_sum(ex, dst, num_segments=n)
    msg = _edge_msg(el, ex, denom[dst])
    out = jax.ops.segment_sum(msg, dst, num_segments=n)
    return out + bias


def kernel(x, edge_index, Wl1, Wr1, att1, b1, g1w, g1b, g1m,
           Wl2, Wr2, att2, b2, g2w, g2b, g2m,
           Wl3, Wr3, att3, b3, g3w, g3b, g3m):
    n = x.shape[0]
    loop = jnp.arange(n, dtype=edge_index.dtype)
    src = jnp.concatenate([edge_index[0], loop])
    dst = jnp.concatenate([edge_index[1], loop])

    h = _gatv2(x, src, dst, Wl1, Wr1, att1, b1, 2, HID)
    h = _norm_gelu(h, g1w, g1b, g1m)
    h = _gatv2(h, src, dst, Wl2, Wr2, att2, b2, 2, HID)
    h = _norm_gelu(h, g2w, g2b, g2m)
    h = _gatv2(h, src, dst, Wl3, Wr3, att3, b3, 1, OUT_CH)
    h = _norm_gelu(h, g3w, g3b, g3m)
    return h.mean(axis=0, keepdims=True)
